# Initial kernel scaffold; baseline (speedup 1.0000x reference)
import functools

import jax
import jax.numpy as jnp
from jax import lax
from jax.experimental import pallas as pl
from jax.experimental.pallas import tpu as pltpu
from jax.experimental.pallas import tpu_sc as plsc

_N = 10000
_D = 128
_E = 320000
_EN = _E + _N
_CW = 144
_NPAD = 10016
_CH = 128
_NSC = 2
_NTS = 16
_NW = _NSC * _NTS
_CHUNKS = 81
_EPT = _CHUNKS * _CH
_EP = _NW * _EPT
_RPT = _NPAD // _NTS

_MESH = plsc.VectorSubcoreMesh(
    core_axis_name="c", subcore_axis_name="s", num_cores=_NSC, num_subcores=_NTS
)


def _tc_prep_body(x_ref, w_ref, as_ref, ad_ref, xp_ref, av_ref, bv_ref, m_ref):
    xp = jnp.dot(x_ref[...], w_ref[...], preferred_element_type=jnp.float32)
    a_s = jnp.sum(xp * as_ref[...], axis=1, keepdims=True)
    a_d = jnp.sum(xp * ad_ref[...], axis=1, keepdims=True)
    av_ref[...] = a_s
    bv_ref[...] = a_d
    mx = jnp.max(a_s) + jnp.max(a_d)
    mx = jnp.where(mx < 0.0, mx * jnp.float32(0.2), mx)
    m_ref[...] = jnp.full((1, 128), mx, jnp.float32)
    xp_ref[...] = jnp.concatenate(
        [xp, jnp.ones((_N, 1), jnp.float32), jnp.zeros((_N, _CW - _D - 1), jnp.float32)],
        axis=1,
    )


_tc_prep = pl.pallas_call(
    _tc_prep_body,
    out_shape=[
        jax.ShapeDtypeStruct((_N, _CW), jnp.float32),
        jax.ShapeDtypeStruct((_N, 1), jnp.float32),
        jax.ShapeDtypeStruct((_N, 1), jnp.float32),
        jax.ShapeDtypeStruct((1, 128), jnp.float32),
    ],
)


@functools.partial(
    pl.kernel,
    out_type=[
        jax.ShapeDtypeStruct((_NSC, _NPAD, _CW), jnp.float32),
        jax.ShapeDtypeStruct((_EP,), jnp.float32),
    ],
    mesh=_MESH,
    scratch_types=[
        pltpu.VMEM((_N,), jnp.float32),
        pltpu.VMEM((_N,), jnp.float32),
        pltpu.VMEM((16,), jnp.float32),
        pltpu.VMEM((_CH,), jnp.int32),
        pltpu.VMEM((_CH,), jnp.int32),
        pltpu.VMEM((_CH, _CW), jnp.float32),
        pltpu.VMEM((_CH,), jnp.float32),
        pltpu.VMEM_SHARED((_NPAD, _CW), jnp.float32),
        pltpu.SemaphoreType.DMA,
    ],
)
def _sc_edges(xp_hbm, src_hbm, dst_hbm, as_hbm, ad_hbm, m_hbm,
              acc_out, p_out,
              as_v, ad_v, m_v, src_v, dst_v, rows_v, p_v, acc_sh, sem):
    cid = lax.axis_index("c")
    sid = lax.axis_index("s")
    wid = cid * _NTS + sid

    pltpu.sync_copy(as_hbm, as_v)
    pltpu.sync_copy(ad_hbm, ad_v)
    pltpu.sync_copy(m_hbm, m_v)

    def _zrow(r, carry):
        for j in range(_CW // 16):
            rows_v[r, pl.ds(j * 16, 16)] = jnp.zeros((16,), jnp.float32)
        return carry

    lax.fori_loop(0, _CH, _zrow, 0)
    base_r = sid * _RPT
    for k in range(_RPT // _CH):
        pltpu.sync_copy(rows_v, acc_sh.at[pl.ds(base_r + k * _CH, _CH)])
    _rem = _RPT % _CH
    pltpu.sync_copy(rows_v.at[pl.ds(0, _rem)],
                    acc_sh.at[pl.ds(base_r + (_RPT // _CH) * _CH, _rem)])
    plsc.subcore_barrier()

    mvec = m_v[...]

    def _chunk(i, carry):
        base = wid * _EPT + i * _CH
        pltpu.sync_copy(src_hbm.at[pl.ds(base, _CH)], src_v)
        pltpu.sync_copy(dst_hbm.at[pl.ds(base, _CH)], dst_v)
        gat = pltpu.async_copy(xp_hbm.at[src_v], rows_v, sem)
        for g in range(_CH // 16):
            sl = pl.ds(g * 16, 16)
            av = plsc.load_gather(as_v, [src_v[sl]])
            bv = plsc.load_gather(ad_v, [dst_v[sl]])
            e = av + bv
            e = jnp.where(e < 0.0, e * jnp.float32(0.2), e)
            p_v[sl] = jnp.exp(e - mvec)
        gat.wait()

        def _edge(r, c2):
            pb = plsc.load_gather(p_v, [jnp.broadcast_to(r, (16,)).astype(jnp.int32)])
            for j in range(_CW // 16):
                sl2 = pl.ds(j * 16, 16)
                rows_v[r, sl2] = rows_v[r, sl2] * pb
            return c2

        lax.fori_loop(0, _CH, _edge, 0, unroll=4)
        pltpu.sync_copy(rows_v, acc_sh.at[dst_v], add=True)
        pltpu.sync_copy(p_v, p_out.at[pl.ds(base, _CH)])
        return carry

    lax.fori_loop(0, _CHUNKS, _chunk, 0)
    plsc.subcore_barrier()

    for k in range(_RPT // _CH):
        sl = pl.ds(base_r + k * _CH, _CH)
        pltpu.sync_copy(acc_sh.at[sl], acc_out.at[cid, sl])
    sl_last = pl.ds(base_r + (_RPT // _CH) * _CH, _rem)
    pltpu.sync_copy(acc_sh.at[sl_last], acc_out.at[cid, sl_last])


def _tc_finish_body(acc_ref, b_ref, h_ref, r_ref):
    s = acc_ref[0] + acc_ref[1]
    den = s[:, _D:_D + 1]
    rr = 1.0 / (den + jnp.float32(1e-16))
    h_ref[...] = s[:, :_D] * rr + b_ref[...]
    r_ref[...] = rr


_tc_finish = pl.pallas_call(
    _tc_finish_body,
    out_shape=[
        jax.ShapeDtypeStruct((_NPAD, _D), jnp.float32),
        jax.ShapeDtypeStruct((_NPAD, 1), jnp.float32),
    ],
)


@functools.partial(
    pl.kernel,
    out_type=jax.ShapeDtypeStruct((_EP,), jnp.float32),
    mesh=_MESH,
    scratch_types=[
        pltpu.VMEM((_NPAD,), jnp.float32),
        pltpu.VMEM((_CH,), jnp.float32),
        pltpu.VMEM((_CH,), jnp.int32),
        pltpu.VMEM((_CH,), jnp.float32),
    ],
)
def _sc_alpha(p_hbm, dst_hbm, r_hbm, alpha_out, r_v, pb_v, db_v, ab_v):
    cid = lax.axis_index("c")
    sid = lax.axis_index("s")
    wid = cid * _NTS + sid
    pltpu.sync_copy(r_hbm, r_v)

    def _chunk(i, carry):
        base = wid * _EPT + i * _CH
        pltpu.sync_copy(p_hbm.at[pl.ds(base, _CH)], pb_v)
        pltpu.sync_copy(dst_hbm.at[pl.ds(base, _CH)], db_v)
        for g in range(_CH // 16):
            sl = pl.ds(g * 16, 16)
            rv = plsc.load_gather(r_v, [db_v[sl]])
            ab_v[sl] = pb_v[sl] * rv
        pltpu.sync_copy(ab_v, alpha_out.at[pl.ds(base, _CH)])
        return carry

    lax.fori_loop(0, _CHUNKS, _chunk, 0)


def kernel(x, edge_index, W, att_src, att_dst, bias):
    xp_pad, a_s, a_d, m = _tc_prep(
        x, W, att_src.reshape(1, _D), att_dst.reshape(1, _D)
    )
    a_s = a_s.reshape(_N)
    a_d = a_d.reshape(_N)
    m16 = m.reshape(128)[:16]

    loop = jnp.arange(_N, dtype=jnp.int32)
    npad_e = _EP - _EN
    src_full = jnp.concatenate(
        [edge_index[0], loop, jnp.zeros((npad_e,), jnp.int32)])
    dst_full = jnp.concatenate(
        [edge_index[1], loop, jnp.full((npad_e,), _N, jnp.int32)])

    acc, p = _sc_edges(xp_pad, src_full, dst_full, a_s, a_d, m16)
    h_full, r = _tc_finish(acc, bias.reshape(1, _D))
    alpha = _sc_alpha(p, dst_full, r.reshape(_NPAD))
    return h_full[:_N], alpha[:_EN].reshape(_EN, 1)


# speedup vs baseline: 19.5458x; 19.5458x over previous
import functools

import jax
import jax.numpy as jnp
from jax import lax
from jax.experimental import pallas as pl
from jax.experimental.pallas import tpu as pltpu
from jax.experimental.pallas import tpu_sc as plsc

_N = 10000
_D = 128
_E = 320000
_EN = _E + _N
_NPAD = 10112
_CH = 128
_NSC = 2
_NTS = 16
_NW = _NSC * _NTS
_CHUNKS = 81
_EPT = _CHUNKS * _CH
_EP = _NW * _EPT
_RPT = _NPAD // _NTS

_MESH = plsc.VectorSubcoreMesh(
    core_axis_name="c", subcore_axis_name="s", num_cores=_NSC, num_subcores=_NTS
)
_SC_PARAMS = pltpu.CompilerParams(needs_layout_passes=False)


def _tc_prep_body(x_ref, w_ref, as_ref, ad_ref, xp_ref, av_ref, bv_ref, m_ref):
    xp = jnp.dot(x_ref[...], w_ref[...], preferred_element_type=jnp.float32)
    a_s = jnp.sum(xp * as_ref[...], axis=1, keepdims=True)
    a_d = jnp.sum(xp * ad_ref[...], axis=1, keepdims=True)
    av_ref[...] = a_s
    bv_ref[...] = a_d
    mx = jnp.max(a_s) + jnp.max(a_d)
    mx = jnp.where(mx < 0.0, mx * jnp.float32(0.2), mx)
    m_ref[...] = jnp.full((1, 128), mx, jnp.float32)
    xp_ref[...] = xp


_tc_prep = pl.pallas_call(
    _tc_prep_body,
    out_shape=[
        jax.ShapeDtypeStruct((_N, _D), jnp.float32),
        jax.ShapeDtypeStruct((_N, 1), jnp.float32),
        jax.ShapeDtypeStruct((_N, 1), jnp.float32),
        jax.ShapeDtypeStruct((1, 128), jnp.float32),
    ],
)


@functools.partial(
    pl.kernel,
    out_type=[
        jax.ShapeDtypeStruct((_NSC, _NPAD, _D), jnp.float32),
        jax.ShapeDtypeStruct((_NPAD,), jnp.float32),
        jax.ShapeDtypeStruct((_NPAD,), jnp.float32),
        jax.ShapeDtypeStruct((_EP,), jnp.float32),
    ],
    mesh=_MESH,
    scratch_types=[
        pltpu.VMEM((_N,), jnp.float32),
        pltpu.VMEM((_N,), jnp.float32),
        pltpu.VMEM((16,), jnp.float32),
        pltpu.VMEM((_CH,), jnp.int32),
        pltpu.VMEM((_CH,), jnp.int32),
        pltpu.VMEM((_CH, _D), jnp.float32),
        pltpu.VMEM((_CH,), jnp.float32),
        pltpu.VMEM_SHARED((_NPAD, _D), jnp.float32),
        pltpu.VMEM_SHARED((_NPAD,), jnp.float32),
        pltpu.SemaphoreType.DMA,
    ],
    compiler_params=_SC_PARAMS,
)
def _sc_edges(xp_hbm, src_hbm, dst_hbm, as_hbm, ad_hbm, m_hbm,
              acc_out, accd_out0, accd_out1, p_out,
              as_v, ad_v, m_v, src_v, dst_v, rows_v, p_v, acc_sh, accd_sh, sem):
    cid = lax.axis_index("c")
    sid = lax.axis_index("s")
    wid = cid * _NTS + sid

    pltpu.sync_copy(as_hbm, as_v)
    pltpu.sync_copy(ad_hbm, ad_v)
    pltpu.sync_copy(m_hbm, m_v)

    def _zrow(r, carry):
        for j in range(_D // 16):
            rows_v[r, pl.ds(j * 16, 16)] = jnp.zeros((16,), jnp.float32)
        return carry

    lax.fori_loop(0, _CH, _zrow, 0)
    for g in range(_CH // 16):
        p_v[pl.ds(g * 16, 16)] = jnp.zeros((16,), jnp.float32)

    base_r = sid * _RPT
    _nfull = _RPT // _CH
    _rem = _RPT % _CH
    for k in range(_nfull):
        pltpu.sync_copy(rows_v, acc_sh.at[pl.ds(base_r + k * _CH, _CH)])
        pltpu.sync_copy(p_v, accd_sh.at[pl.ds(base_r + k * _CH, _CH)])
    pltpu.sync_copy(rows_v.at[pl.ds(0, _rem)],
                    acc_sh.at[pl.ds(base_r + _nfull * _CH, _rem)])
    pltpu.sync_copy(p_v.at[pl.ds(0, _rem)],
                    accd_sh.at[pl.ds(base_r + _nfull * _CH, _rem)])
    plsc.subcore_barrier()

    mvec = m_v[...]

    def _chunk(i, carry):
        base = wid * _EPT + i * _CH
        pltpu.sync_copy(src_hbm.at[pl.ds(base, _CH)], src_v)
        pltpu.sync_copy(dst_hbm.at[pl.ds(base, _CH)], dst_v)
        gat = pltpu.async_copy(xp_hbm.at[src_v], rows_v, sem)
        for g in range(_CH // 16):
            sl = pl.ds(g * 16, 16)
            av = plsc.load_gather(as_v, [src_v[sl]])
            bv = plsc.load_gather(ad_v, [dst_v[sl]])
            e = av + bv
            e = jnp.where(e < 0.0, e * jnp.float32(0.2), e)
            p_v[sl] = jnp.exp(e - mvec)
        gat.wait()

        def _edge(r, c2):
            pb = plsc.load_gather(p_v, [jnp.broadcast_to(r, (16,)).astype(jnp.int32)])
            for j in range(_D // 16):
                sl2 = pl.ds(j * 16, 16)
                rows_v[r, sl2] = rows_v[r, sl2] * pb
            return c2

        lax.fori_loop(0, _CH, _edge, 0, unroll=4)
        pltpu.sync_copy(rows_v, acc_sh.at[dst_v], add=True)
        pltpu.sync_copy(p_v, accd_sh.at[dst_v], add=True)
        pltpu.sync_copy(p_v, p_out.at[pl.ds(base, _CH)])
        return carry

    lax.fori_loop(0, _CHUNKS, _chunk, 0)
    plsc.subcore_barrier()

    sl_last = pl.ds(base_r + _nfull * _CH, _rem)
    rem_sl = pl.ds(0, _rem)
    for k in range(_nfull):
        sl = pl.ds(base_r + k * _CH, _CH)
        pltpu.sync_copy(acc_sh.at[sl], rows_v)
        pltpu.sync_copy(rows_v, acc_out.at[cid, sl])
    pltpu.sync_copy(acc_sh.at[sl_last], rows_v.at[rem_sl])
    pltpu.sync_copy(rows_v.at[rem_sl], acc_out.at[cid, sl_last])

    @pl.when(cid == 0)
    def _():
        for k in range(_nfull):
            sl = pl.ds(base_r + k * _CH, _CH)
            pltpu.sync_copy(accd_sh.at[sl], p_v)
            pltpu.sync_copy(p_v, accd_out0.at[sl])
        pltpu.sync_copy(accd_sh.at[sl_last], p_v.at[rem_sl])
        pltpu.sync_copy(p_v.at[rem_sl], accd_out0.at[sl_last])

    @pl.when(cid == 1)
    def _():
        for k in range(_nfull):
            sl = pl.ds(base_r + k * _CH, _CH)
            pltpu.sync_copy(accd_sh.at[sl], p_v)
            pltpu.sync_copy(p_v, accd_out1.at[sl])
        pltpu.sync_copy(accd_sh.at[sl_last], p_v.at[rem_sl])
        pltpu.sync_copy(p_v.at[rem_sl], accd_out1.at[sl_last])


def _tc_finish_body(acc_ref, accd_ref, b_ref, h_ref, r_ref):
    s = acc_ref[0] + acc_ref[1]
    den = accd_ref[0] + accd_ref[1]
    rr = 1.0 / (den + jnp.float32(1e-16))
    h_ref[...] = s * rr + b_ref[...]
    r_ref[...] = rr


_tc_finish = pl.pallas_call(
    _tc_finish_body,
    out_shape=[
        jax.ShapeDtypeStruct((_NPAD, _D), jnp.float32),
        jax.ShapeDtypeStruct((_NPAD, 1), jnp.float32),
    ],
)


@functools.partial(
    pl.kernel,
    out_type=jax.ShapeDtypeStruct((_EP,), jnp.float32),
    mesh=_MESH,
    scratch_types=[
        pltpu.VMEM((_NPAD,), jnp.float32),
        pltpu.VMEM((_CH,), jnp.float32),
        pltpu.VMEM((_CH,), jnp.int32),
        pltpu.VMEM((_CH,), jnp.float32),
    ],
    compiler_params=_SC_PARAMS,
)
def _sc_alpha(p_hbm, dst_hbm, r_hbm, alpha_out, r_v, pb_v, db_v, ab_v):
    cid = lax.axis_index("c")
    sid = lax.axis_index("s")
    wid = cid * _NTS + sid
    pltpu.sync_copy(r_hbm, r_v)

    def _chunk(i, carry):
        base = wid * _EPT + i * _CH
        pltpu.sync_copy(p_hbm.at[pl.ds(base, _CH)], pb_v)
        pltpu.sync_copy(dst_hbm.at[pl.ds(base, _CH)], db_v)
        for g in range(_CH // 16):
            sl = pl.ds(g * 16, 16)
            rv = plsc.load_gather(r_v, [db_v[sl]])
            ab_v[sl] = pb_v[sl] * rv
        pltpu.sync_copy(ab_v, alpha_out.at[pl.ds(base, _CH)])
        return carry

    lax.fori_loop(0, _CHUNKS, _chunk, 0)


def kernel(x, edge_index, W, att_src, att_dst, bias):
    xp, a_s, a_d, m = _tc_prep(
        x, W, att_src.reshape(1, _D), att_dst.reshape(1, _D)
    )
    a_s = a_s.reshape(_N)
    a_d = a_d.reshape(_N)
    m16 = m.reshape(128)[:16]

    loop = jnp.arange(_N, dtype=jnp.int32)
    npad_e = _EP - _EN
    src_full = jnp.concatenate(
        [edge_index[0], loop, jnp.zeros((npad_e,), jnp.int32)])
    dst_full = jnp.concatenate(
        [edge_index[1], loop, jnp.full((npad_e,), _N, jnp.int32)])

    acc, accd0, accd1, p = _sc_edges(xp, src_full, dst_full, a_s, a_d, m16)
    accd = jnp.stack([accd0, accd1]).reshape(_NSC, _NPAD, 1)
    h_full, r = _tc_finish(acc, accd, bias.reshape(1, _D))
    alpha = _sc_alpha(p, dst_full, r.reshape(_NPAD))
    return h_full[:_N], alpha[:_EN].reshape(_EN, 1)


# speedup vs baseline: 28.6567x; 1.4661x over previous
import functools

import jax
import jax.numpy as jnp
from jax import lax
from jax.experimental import pallas as pl
from jax.experimental.pallas import tpu as pltpu
from jax.experimental.pallas import tpu_sc as plsc

_N = 10000
_D = 128
_E = 320000
_EN = _E + _N
_AR = 10008
_CH = 112
_NSC = 2
_NTS = 16
_NW = _NSC * _NTS
_CHUNKS = 93
_EPT = _CHUNKS * _CH
_EP = _NW * _EPT
_RPT = 624
_G16 = _CH // 16
_V16 = _D // 16
_NPD = 10016

_MESH = plsc.VectorSubcoreMesh(
    core_axis_name="c", subcore_axis_name="s", num_cores=_NSC, num_subcores=_NTS
)
_SC_PARAMS = pltpu.CompilerParams(needs_layout_passes=False)


def _tc_prep_body(x_ref, w_ref, as_ref, ad_ref, xp_ref, av_ref, bv_ref, m_ref):
    xp = jnp.dot(x_ref[...], w_ref[...], preferred_element_type=jnp.float32)
    a_s = jnp.sum(xp * as_ref[...], axis=1, keepdims=True)
    a_d = jnp.sum(xp * ad_ref[...], axis=1, keepdims=True)
    av_ref[...] = a_s
    bv_ref[...] = a_d
    mx = jnp.max(a_s) + jnp.max(a_d)
    mx = jnp.where(mx < 0.0, mx * jnp.float32(0.2), mx)
    m_ref[...] = jnp.full((1, 128), mx, jnp.float32)
    xp_ref[...] = xp


_tc_prep = pl.pallas_call(
    _tc_prep_body,
    out_shape=[
        jax.ShapeDtypeStruct((_N, _D), jnp.float32),
        jax.ShapeDtypeStruct((_N, 1), jnp.float32),
        jax.ShapeDtypeStruct((_N, 1), jnp.float32),
        jax.ShapeDtypeStruct((1, 128), jnp.float32),
    ],
)


@functools.partial(
    pl.kernel,
    out_type=[
        jax.ShapeDtypeStruct((_NSC, _AR, _D), jnp.float32),
        jax.ShapeDtypeStruct((_AR,), jnp.float32),
        jax.ShapeDtypeStruct((_AR,), jnp.float32),
        jax.ShapeDtypeStruct((_EP,), jnp.float32),
    ],
    mesh=_MESH,
    scratch_types=[
        pltpu.VMEM((16,), jnp.float32),
        pltpu.VMEM((3, _CH), jnp.int32),
        pltpu.VMEM((3, _CH), jnp.int32),
        pltpu.VMEM((3, _CH), jnp.int32),
        pltpu.VMEM((3, _CH), jnp.float32),
        pltpu.VMEM((3, _CH), jnp.float32),
        pltpu.VMEM((3, _CH), jnp.float32),
        pltpu.VMEM((_CH, _D), jnp.float32),
        pltpu.VMEM((_CH, _D), jnp.float32),
        pltpu.VMEM((_CH, _D), jnp.float32),
        pltpu.VMEM_SHARED((_AR, _D), jnp.float32),
        pltpu.VMEM_SHARED((_AR,), jnp.float32),
        pltpu.SemaphoreType.DMA,
        pltpu.SemaphoreType.DMA,
        pltpu.SemaphoreType.DMA,
        pltpu.SemaphoreType.DMA,
        pltpu.SemaphoreType.DMA,
        pltpu.SemaphoreType.DMA,
        pltpu.SemaphoreType.DMA,
        pltpu.SemaphoreType.DMA,
        pltpu.SemaphoreType.DMA,
        pltpu.SemaphoreType.DMA,
        pltpu.SemaphoreType.DMA,
        pltpu.SemaphoreType.DMA,
        pltpu.SemaphoreType.DMA,
        pltpu.SemaphoreType.DMA,
        pltpu.SemaphoreType.DMA,
        pltpu.SemaphoreType.DMA,
    ],
    compiler_params=_SC_PARAMS,
)
def _sc_edges(xp_hbm, src_hbm, dst_hbm, as_hbm, ad_hbm, m_hbm,
              acc_out, accd_out0, accd_out1, p_out,
              m_v, srcv, dstv, dscv, asg, adg, pv,
              rows0_v, rows1_v, rows2_v, acc_sh, accd_sh,
              lsem, isem0, isem1, isem2, gsem0, gsem1, gsem2,
              ssem0, ssem1, ssem2, dsem0, dsem1, dsem2,
              psem0, psem1, psem2):
    cid = lax.axis_index("c")
    sid = lax.axis_index("s")
    wid = cid * _NTS + sid
    rows = (rows0_v, rows1_v, rows2_v)
    isem = (isem0, isem1, isem2)
    gsem = (gsem0, gsem1, gsem2)
    ssem = (ssem0, ssem1, ssem2)
    dsem = (dsem0, dsem1, dsem2)
    psem = (psem0, psem1, psem2)

    ld_m = pltpu.async_copy(m_hbm, m_v, lsem)

    def _zrow(r, carry):
        for j in range(_V16):
            rows0_v[r, pl.ds(j * 16, 16)] = jnp.zeros((16,), jnp.float32)
        return carry

    lax.fori_loop(0, _CH, _zrow, 0)
    for g in range(_G16):
        pv[0, pl.ds(g * 16, 16)] = jnp.zeros((16,), jnp.float32)

    base_r = sid * _RPT
    _zchunks = [(0, _CH), (_CH, _CH), (2 * _CH, _CH), (3 * _CH, _CH),
                (4 * _CH, _CH), (5 * _CH, _RPT - 5 * _CH)]

    def _zero_range(r0, chunks):
        for off, ln in chunks:
            pltpu.sync_copy(rows0_v.at[pl.ds(0, ln)],
                            acc_sh.at[pl.ds(r0 + off, ln)])
            pltpu.sync_copy(pv.at[0, pl.ds(0, ln)],
                            accd_sh.at[pl.ds(r0 + off, ln)])

    _zero_range(base_r, _zchunks)

    @pl.when(sid == _NTS - 1)
    def _():
        _zero_range(_NTS * _RPT, [(0, _AR - _NTS * _RPT)])

    plsc.subcore_barrier()
    ld_m.wait()
    mvec = m_v[...]

    ebase = wid * _EPT

    def _load_idx(c, b):
        pltpu.async_copy(src_hbm.at[pl.ds(ebase + c * _CH, _CH)], srcv.at[b],
                         isem[b])
        pltpu.async_copy(dst_hbm.at[pl.ds(ebase + c * _CH, _CH)], dstv.at[b],
                         isem[b])

    def _wait_idx(c, b):
        pltpu.make_async_copy(src_hbm.at[pl.ds(0, _CH)], srcv.at[b],
                              isem[b]).wait()
        pltpu.make_async_copy(dst_hbm.at[pl.ds(0, _CH)], dstv.at[b],
                              isem[b]).wait()

    def _start_gather(b):
        pltpu.async_copy(xp_hbm.at[srcv.at[b]], rows[b], gsem[b])
        pltpu.async_copy(as_hbm.at[srcv.at[b]], asg.at[b], gsem[b])
        pltpu.async_copy(ad_hbm.at[dstv.at[b]], adg.at[b], gsem[b])

    def _wait_gather(b):
        pltpu.make_async_copy(xp_hbm.at[pl.ds(0, _CH)], rows[b],
                              gsem[b]).wait()
        pltpu.make_async_copy(as_hbm.at[pl.ds(0, _CH)], asg.at[b],
                              gsem[b]).wait()
        pltpu.make_async_copy(ad_hbm.at[pl.ds(0, _CH)], adg.at[b],
                              gsem[b]).wait()

    def _start_scatter(c, b):
        pltpu.async_copy(rows[b], acc_sh.at[dscv.at[b]], ssem[b], add=True)
        pltpu.async_copy(pv.at[b], accd_sh.at[dscv.at[b]], dsem[b], add=True)
        pltpu.async_copy(pv.at[b], p_out.at[pl.ds(ebase + c * _CH, _CH)],
                         psem[b])

    def _wait_scatter(c, b):
        pltpu.make_async_copy(xp_hbm.at[pl.ds(0, _CH)], rows[b],
                              ssem[b]).wait()
        pltpu.make_async_copy(as_hbm.at[pl.ds(0, _CH)], pv.at[b],
                              dsem[b]).wait()
        pltpu.make_async_copy(pv.at[b], p_out.at[pl.ds(0, _CH)],
                              psem[b]).wait()

    def _compute(c, b):
        for g in range(_G16):
            sl = pl.ds(g * 16, 16)
            e = asg[b, sl] + adg[b, sl]
            e = jnp.where(e < 0.0, e * jnp.float32(0.2), e)
            pv[b, sl] = jnp.exp(e - mvec)
            dscv[b, sl] = dstv[b, sl]

        def _edge(r, c2):
            rb = jnp.broadcast_to(r, (16,)).astype(jnp.int32)
            pb = plsc.load_gather(pv.at[b], [rb])
            for j in range(_V16):
                sl2 = pl.ds(j * 16, 16)
                rows[b][r, sl2] = rows[b][r, sl2] * pb
            return c2

        lax.fori_loop(0, _CH, _edge, 0, unroll=4)

    def _slot(c, b, wait_prev=True, load_next2=True, start_next=True):
        bn = (b + 1) % 3
        bnn = (b + 2) % 3
        if wait_prev:
            _wait_scatter(c - 2, bn)
        if start_next:
            _wait_idx(c + 1, bn)
            _start_gather(bn)
        if load_next2:
            _load_idx(c + 2, bnn)
        _wait_gather(b)
        _compute(c, b)
        _start_scatter(c, b)

    _load_idx(jnp.int32(0), 0)
    _wait_idx(jnp.int32(0), 0)
    _start_gather(0)
    _load_idx(jnp.int32(1), 1)
    _slot(jnp.int32(0), 0, wait_prev=False)
    _slot(jnp.int32(1), 1, wait_prev=False)

    def _triple(j, carry):
        c = 3 * j + 2
        _slot(c, 2)
        _slot(c + 1, 0)
        _slot(c + 2, 1)
        return carry

    lax.fori_loop(0, (_CHUNKS - 4) // 3, _triple, 0)

    _slot(jnp.int32(_CHUNKS - 4), 2)
    _slot(jnp.int32(_CHUNKS - 3), 0)
    _slot(jnp.int32(_CHUNKS - 2), 1, load_next2=False)
    _slot(jnp.int32(_CHUNKS - 1), 2, load_next2=False,
          start_next=False)
    _wait_scatter(jnp.int32(_CHUNKS - 2), 1)
    _wait_scatter(jnp.int32(_CHUNKS - 1), 2)
    plsc.subcore_barrier()

    def _copy_range(r0, chunks, accd_dst):
        for off, ln in chunks:
            pltpu.sync_copy(acc_sh.at[pl.ds(r0 + off, ln)],
                            rows0_v.at[pl.ds(0, ln)])
            pltpu.sync_copy(rows0_v.at[pl.ds(0, ln)],
                            acc_out.at[cid, pl.ds(r0 + off, ln)])
            pltpu.sync_copy(accd_sh.at[pl.ds(r0 + off, ln)],
                            pv.at[0, pl.ds(0, ln)])
            pltpu.sync_copy(pv.at[0, pl.ds(0, ln)],
                            accd_dst.at[pl.ds(r0 + off, ln)])

    @pl.when(cid == 0)
    def _():
        _copy_range(base_r, _zchunks, accd_out0)

        @pl.when(sid == _NTS - 1)
        def _():
            _copy_range(_NTS * _RPT, [(0, _AR - _NTS * _RPT)], accd_out0)

    @pl.when(cid == 1)
    def _():
        _copy_range(base_r, _zchunks, accd_out1)

        @pl.when(sid == _NTS - 1)
        def _():
            _copy_range(_NTS * _RPT, [(0, _AR - _NTS * _RPT)], accd_out1)


def _tc_finish_body(acc_ref, accd_ref, b_ref, h_ref):
    s = acc_ref[0] + acc_ref[1]
    den = accd_ref[0] + accd_ref[1]
    rr = 1.0 / (den + jnp.float32(1e-16))
    h_ref[...] = s * rr + b_ref[...]


_tc_finish = pl.pallas_call(
    _tc_finish_body,
    out_shape=jax.ShapeDtypeStruct((_AR, _D), jnp.float32),
)


@functools.partial(
    pl.kernel,
    out_type=jax.ShapeDtypeStruct((_EP,), jnp.float32),
    mesh=_MESH,
    scratch_types=[
        pltpu.VMEM((_NPD,), jnp.float32),
        pltpu.VMEM((_NPD,), jnp.float32),
        pltpu.VMEM((_EPT,), jnp.float32),
        pltpu.VMEM((_EPT,), jnp.int32),
        pltpu.VMEM((_EPT,), jnp.float32),
        pltpu.SemaphoreType.DMA,
    ],
    compiler_params=_SC_PARAMS,
)
def _sc_alpha(p_hbm, dst_hbm, accd0_hbm, accd1_hbm, alpha_out,
              r_v, d1_v, pall_v, dstall_v, aall_v, lsem):
    cid = lax.axis_index("c")
    sid = lax.axis_index("s")
    wid = cid * _NTS + sid
    ebase = wid * _EPT
    lds = [
        pltpu.async_copy(accd0_hbm, r_v.at[pl.ds(0, _AR)], lsem),
        pltpu.async_copy(accd1_hbm, d1_v.at[pl.ds(0, _AR)], lsem),
        pltpu.async_copy(p_hbm.at[pl.ds(ebase, _EPT)], pall_v, lsem),
        pltpu.async_copy(dst_hbm.at[pl.ds(ebase, _EPT)], dstall_v, lsem),
    ]
    for h in lds:
        h.wait()

    def _rinit(i, carry):
        sl = pl.ds(i * 16, 16)
        r_v[sl] = 1.0 / (r_v[sl] + d1_v[sl] + jnp.float32(1e-16))
        return carry

    lax.fori_loop(0, _NPD // 16, _rinit, 0)

    def _grp(i, carry):
        sl = pl.ds(i * 16, 16)
        rv = plsc.load_gather(r_v, [dstall_v[sl]])
        aall_v[sl] = pall_v[sl] * rv
        return carry

    lax.fori_loop(0, _EPT // 16, _grp, 0, unroll=4)
    pltpu.sync_copy(aall_v, alpha_out.at[pl.ds(ebase, _EPT)])


def kernel(x, edge_index, W, att_src, att_dst, bias):
    xp, a_s, a_d, m = _tc_prep(
        x, W, att_src.reshape(1, _D), att_dst.reshape(1, _D)
    )
    a_s = a_s.reshape(_N)
    a_d = jnp.concatenate([a_d.reshape(_N), jnp.zeros((_NPD - _N,), jnp.float32)])
    m16 = m.reshape(128)[:16]

    loop = jnp.arange(_N, dtype=jnp.int32)
    npad_e = _EP - _EN
    src_full = jnp.concatenate(
        [edge_index[0], loop, jnp.zeros((npad_e,), jnp.int32)])
    dst_full = jnp.concatenate(
        [edge_index[1], loop, jnp.full((npad_e,), _N, jnp.int32)])

    acc, accd0, accd1, p = _sc_edges(xp, src_full, dst_full, a_s, a_d, m16)
    accd = jnp.stack([accd0, accd1]).reshape(_NSC, _AR, 1)
    h_full = _tc_finish(acc, accd, bias.reshape(1, _D))
    alpha = _sc_alpha(p, dst_full, accd0, accd1)
    return h_full[:_N], alpha[:_EN].reshape(_EN, 1)
